# Initial kernel scaffold; baseline (speedup 1.0000x reference)
import jax
import jax.numpy as jnp
from jax.experimental import pallas as pl


def kernel(x, edge_index, bank, node1, node2, W1, b1, W2, b2, Wb, bb, Wf1, bf1, Wf2, bf2, Wf3, bf3):
    raise NotImplementedError("write your pallas kernel here")


# speedup vs baseline: 2.5717x; 2.5717x over previous
import functools
import math

import jax
import jax.numpy as jnp
from jax import lax
from jax.experimental import pallas as pl
from jax.experimental.pallas import tpu as pltpu

N_NODES = 100000
N_BANK = 10000
N_PAIRS = 500000
FW = 16


def _matmul_body(x_ref, w_ref, c_ref, o_ref):
    o_ref[...] = (
        jnp.dot(x_ref[...], w_ref[...], preferred_element_type=jnp.float32)
        + c_ref[...]
    )


def _node_matmul(x32, w_pad, row_const, bm):
    m = x32.shape[0]
    grid = (m + bm - 1) // bm
    return pl.pallas_call(
        _matmul_body,
        grid=(grid,),
        in_specs=[
            pl.BlockSpec((bm, 32), lambda i: (i, 0)),
            pl.BlockSpec((32, FW), lambda i: (0, 0)),
            pl.BlockSpec((1, FW), lambda i: (0, 0)),
        ],
        out_specs=pl.BlockSpec((bm, FW), lambda i: (i, 0)),
        out_shape=jax.ShapeDtypeStruct((m, FW), jnp.float32),
    )(x32, w_pad, row_const)


def _softmax_body(z_ref, o_ref, m_ref, s_ref):
    phase = pl.program_id(0)
    nb = pl.num_programs(1)
    i = pl.program_id(1)

    @pl.when(jnp.logical_and(phase == 0, i == 0))
    def _init():
        m_ref[...] = jnp.full_like(m_ref, -jnp.inf)
        s_ref[...] = jnp.zeros_like(s_ref)

    @pl.when(phase == 0)
    def _acc():
        blk = z_ref[...]
        bm = jnp.max(blk, axis=0, keepdims=True)
        m_old = m_ref[...]
        m_new = jnp.maximum(m_old, bm)
        bs = jnp.sum(jnp.exp(blk - m_new), axis=0, keepdims=True)
        s_ref[...] = s_ref[...] * jnp.exp(m_old - m_new) + bs
        m_ref[...] = m_new

        @pl.when(i == nb - 1)
        def _lanefold():
            mm = m_ref[...]
            ss = s_ref[...]
            for k in (4, 8, 16, 32, 64):
                mr = pltpu.roll(mm, k, 1)
                sr = pltpu.roll(ss, k, 1)
                mn = jnp.maximum(mm, mr)
                ss = ss * jnp.exp(mm - mn) + sr * jnp.exp(mr - mn)
                mm = mn
            m_ref[...] = mm
            s_ref[...] = ss

    @pl.when(phase == 1)
    def _emit():
        o_ref[...] = jnp.exp(z_ref[...] - m_ref[...]) * (1.0 / s_ref[...])


def _softmax_axis0(z2d, bm):
    r = z2d.shape[0]
    grid = (r + bm - 1) // bm
    return pl.pallas_call(
        _softmax_body,
        grid=(2, grid),
        in_specs=[pl.BlockSpec((bm, 128), lambda p, i: (i, 0))],
        out_specs=pl.BlockSpec((bm, 128), lambda p, i: (i, 0)),
        out_shape=jax.ShapeDtypeStruct((r, 128), jnp.float32),
        scratch_shapes=[
            pltpu.VMEM((1, 128), jnp.float32),
            pltpu.VMEM((1, 128), jnp.float32),
        ],
        compiler_params=pltpu.CompilerParams(
            dimension_semantics=("arbitrary", "arbitrary")
        ),
    )(z2d)


def kernel(x, edge_index, bank, node1, node2, W1, b1, W2, b2, Wb, bb,
           Wf1, bf1, Wf2, bf2, Wf3, bf3):
    f32 = jnp.float32
    src = edge_index[0].astype(jnp.int32)
    dst = edge_index[1].astype(jnp.int32)
    n1 = node1.astype(jnp.int32)
    n2 = node2.astype(jnp.int32)

    Wc = Wf1 @ Wf2 @ Wf3
    bc = bf1 @ Wf2 @ Wf3 + bf2 @ Wf3 + bf3
    Wcc = jnp.concatenate([Wc[:50], Wc[50:]], axis=1)
    W_uv = W1 @ W2 @ Wcc
    c1 = b1 @ W2
    cuv = c1 @ Wcc
    duv = b2 @ Wcc
    bank_w = Wb @ Wcc
    bank_b = bb @ Wcc

    x32 = jnp.pad(x, ((0, 0), (0, 5)))
    w_pad = jnp.pad(W_uv, ((0, 5), (0, FW - 8)))
    ones_row = jnp.zeros((1, FW), f32).at[0, 8].set(1.0)
    g = _node_matmul(x32, w_pad, ones_row, 5000)

    bank32 = jnp.pad(bank, ((0, 0), (0, 5)))
    bw_pad = jnp.pad(bank_w, ((0, 5), (0, FW - 8)))
    bank_uv = _node_matmul(bank32, bw_pad, jnp.pad(bank_b, (0, FW - 8))[None, :], 5000)

    deg = jnp.zeros((N_NODES,), f32).at[dst].add(1.0) + 1.0
    dis = deg ** -0.5

    def prop(t):
        ts = t * dis[:, None]
        acc = jnp.zeros_like(t).at[dst].add(ts[src])
        return dis[:, None] * acc + (dis * dis)[:, None] * t

    q1 = prop(g)
    r = q1[:, 8:9]
    q2 = prop(q1)

    uv = q2[:, :8] + r * cuv[None, :] + duv[None, :]
    bchalf = jnp.concatenate([bc, jnp.zeros((4,), f32)])
    uv_full = jnp.concatenate([uv + bchalf[None, :],
                               bank_uv[:, :8] + bchalf[None, :]], axis=0)

    z = uv_full[n1, :4] + uv_full[n2, 4:8]

    rrows = N_PAIRS // 32
    bm = 1024
    rpad = ((rrows + bm - 1) // bm) * bm
    z2d = jnp.pad(z.reshape(rrows, 128), ((0, rpad - rrows), (0, 0)),
                  constant_values=-jnp.inf)
    out2d = _softmax_axis0(z2d, bm)
    return out2d[:rrows].reshape(N_PAIRS, 4)


# speedup vs baseline: 17.5081x; 6.8079x over previous
import functools

import jax
import jax.numpy as jnp
from jax import lax
from jax.experimental import pallas as pl
from jax.experimental.pallas import tpu as pltpu
from jax.experimental.pallas import tpu_sc as plsc

N_NODES = 100000
N_BANK = 10000
N_PAIRS = 500000
N_EDGES = 1600000
FW = 16
NC, NS = 2, 16
NW = NC * NS
NR = 100352
STRIPE = NR // NS
DUMMY_NODE = N_NODES
ECH = 12800
EPAD = ECH * 128
CW = ECH // NW
NPF = 110016
DUMMY_PAIR = N_NODES + N_BANK
PCH = 4096
PPAD = PCH * 128
PCW = PCH // NW

_sc_mesh = plsc.VectorSubcoreMesh(core_axis_name="c", subcore_axis_name="s")
_f32 = jnp.float32


def _matmul_body(x_ref, w_ref, c_ref, o_ref):
    o_ref[...] = (
        jnp.dot(x_ref[...], w_ref[...], preferred_element_type=_f32)
        + c_ref[...]
    )


def _node_matmul(x32, w_pad, row_const, bm):
    m = x32.shape[0]
    return pl.pallas_call(
        _matmul_body,
        grid=(m // bm,),
        in_specs=[
            pl.BlockSpec((bm, 32), lambda i: (i, 0)),
            pl.BlockSpec((32, FW), lambda i: (0, 0)),
            pl.BlockSpec((1, FW), lambda i: (0, 0)),
        ],
        out_specs=pl.BlockSpec((bm, FW), lambda i: (i, 0)),
        out_shape=jax.ShapeDtypeStruct((m, FW), _f32),
    )(x32, w_pad, row_const)


def _scale1_body(d0_ref, d1_ref, g_ref, ts_ref, dis_ref):
    dis = lax.rsqrt(d0_ref[...] + d1_ref[...] + 1.0)
    dis_ref[...] = dis
    ts_ref[...] = dis * g_ref[...]


def _scale2_body(a0_ref, a1_ref, g_ref, dis_ref, out1_ref, ts_ref):
    dis = dis_ref[...]
    out1 = dis * (a0_ref[...] + a1_ref[...]) + (dis * dis) * g_ref[...]
    out1_ref[...] = out1
    ts_ref[...] = dis * out1


def _scale3_body(a0_ref, a1_ref, out1_ref, dis_ref, cuv_ref, duv_ref, q_ref):
    dis = dis_ref[...]
    out1 = out1_ref[...]
    out2 = dis * (a0_ref[...] + a1_ref[...]) + (dis * dis) * out1
    r = out1[:, 8:9]
    q_ref[...] = out2 + r * cuv_ref[...] + duv_ref[...]


def _row_kernel(body, n_out, inputs, col_inputs=0, bm=STRIPE):
    m = inputs[0].shape[0]
    specs = []
    for a in inputs:
        if a.shape == (m, FW):
            specs.append(pl.BlockSpec((bm, FW), lambda i: (i, 0)))
        elif a.shape == (m, 1):
            specs.append(pl.BlockSpec((bm, 1), lambda i: (i, 0)))
        else:
            specs.append(pl.BlockSpec((1, FW), lambda i: (0, 0)))
    out_shapes = []
    out_specs = []
    for shp in n_out:
        out_shapes.append(jax.ShapeDtypeStruct((m, shp), _f32))
        out_specs.append(pl.BlockSpec((bm, shp), lambda i: (i, 0)))
    return pl.pallas_call(
        body,
        grid=(m // bm,),
        in_specs=specs,
        out_specs=out_specs,
        out_shape=out_shapes,
    )(*inputs)


def _softmax_body(a_ref, b_ref, o_ref, m_ref, s_ref):
    phase = pl.program_id(0)
    nb = pl.num_programs(1)
    i = pl.program_id(1)

    @pl.when(jnp.logical_and(phase == 0, i == 0))
    def _init():
        m_ref[...] = jnp.full_like(m_ref, -jnp.inf)
        s_ref[...] = jnp.zeros_like(s_ref)

    @pl.when(phase == 0)
    def _acc():
        blk = a_ref[...] + b_ref[...]
        bm_ = jnp.max(blk, axis=0, keepdims=True)
        m_old = m_ref[...]
        m_new = jnp.maximum(m_old, bm_)
        bs = jnp.sum(jnp.exp(blk - m_new), axis=0, keepdims=True)
        s_ref[...] = s_ref[...] * jnp.exp(m_old - m_new) + bs
        m_ref[...] = m_new

        @pl.when(i == nb - 1)
        def _lanefold():
            mm = m_ref[...]
            ss = s_ref[...]
            for k in (16, 32, 64):
                mr = pltpu.roll(mm, k, 1)
                sr = pltpu.roll(ss, k, 1)
                mn = jnp.maximum(mm, mr)
                ss = ss * jnp.exp(mm - mn) + sr * jnp.exp(mr - mn)
                mm = mn
            m_ref[...] = mm
            s_ref[...] = ss

    @pl.when(phase == 1)
    def _emit():
        o_ref[...] = jnp.exp(a_ref[...] + b_ref[...] - m_ref[...]) * (1.0 / s_ref[...])


def _softmax_axis0(a2d, b2d, bm):
    r = a2d.shape[0]
    spec = pl.BlockSpec((bm, 128), lambda p, i: (i, 0))
    return pl.pallas_call(
        _softmax_body,
        grid=(2, r // bm),
        in_specs=[spec, spec],
        out_specs=pl.BlockSpec((bm, 128), lambda p, i: (i, 0)),
        out_shape=jax.ShapeDtypeStruct((r, 128), _f32),
        scratch_shapes=[
            pltpu.VMEM((1, 128), _f32),
            pltpu.VMEM((1, 128), _f32),
        ],
        compiler_params=pltpu.CompilerParams(
            dimension_semantics=("arbitrary", "arbitrary")
        ),
    )(a2d, b2d)


@functools.partial(
    pl.kernel,
    out_type=jax.ShapeDtypeStruct((NC, NR), _f32),
    mesh=_sc_mesh,
    compiler_params=pltpu.CompilerParams(use_tc_tiling_on_sc=False),
    scratch_types=[
        pltpu.VMEM((CW, 128), jnp.int32),
        pltpu.VMEM((128,), _f32),
        pltpu.VMEM((STRIPE,), _f32),
        pltpu.VMEM_SHARED((NR,), _f32),
        pltpu.SemaphoreType.DMA,
    ],
)
def _deg_kernel(dst_hbm, out_hbm, didx, ones_v, zer_v, acc, sem):
    c = lax.axis_index("c")
    s = lax.axis_index("s")
    w = s * NC + c

    @pl.loop(0, STRIPE // 16)
    def _zf(i):
        zer_v[pl.ds(i * 16, 16)] = jnp.zeros((16,), _f32)

    for i in range(8):
        ones_v[pl.ds(i * 16, 16)] = jnp.ones((16,), _f32)

    pltpu.sync_copy(zer_v, acc.at[pl.ds(s * STRIPE, STRIPE)])
    pltpu.sync_copy(dst_hbm.at[pl.ds(w * CW, CW)], didx)
    plsc.subcore_barrier()

    @pl.loop(0, CW // 8)
    def _grp(g):
        descs = [
            pltpu.async_copy(ones_v, acc.at[didx.at[g * 8 + j]], sem, add=True)
            for j in range(8)
        ]
        for d in descs:
            d.wait()

    plsc.subcore_barrier()
    pltpu.sync_copy(acc.at[pl.ds(s * STRIPE, STRIPE)],
                    out_hbm.at[c, pl.ds(s * STRIPE, STRIPE)])


@functools.partial(
    pl.kernel,
    out_type=jax.ShapeDtypeStruct((NC, NR, FW), _f32),
    mesh=_sc_mesh,
    compiler_params=pltpu.CompilerParams(use_tc_tiling_on_sc=False),
    scratch_types=[
        pltpu.VMEM((8, 128), jnp.int32),
        pltpu.VMEM((8, 128), jnp.int32),
        pltpu.VMEM((8, 128, FW), _f32),
        pltpu.VMEM((128, FW), _f32),
        pltpu.VMEM_SHARED((NR, FW), _f32),
        pltpu.SemaphoreType.DMA,
        pltpu.SemaphoreType.DMA,
    ],
)
def _prop_kernel(src_hbm, dst_hbm, ts_hbm, out_hbm,
                 sidx, didx, rows, zer, acc, sem1, sem2):
    c = lax.axis_index("c")
    s = lax.axis_index("s")
    w = s * NC + c

    @pl.loop(0, 128)
    def _zf(i):
        zer[i, :] = jnp.zeros((16,), _f32)

    @pl.loop(0, STRIPE // 128)
    def _zs(k):
        pltpu.sync_copy(zer, acc.at[pl.ds(s * STRIPE + k * 128, 128)])

    plsc.subcore_barrier()

    @pl.loop(0, CW // 8)
    def _grp(g):
        base = w * CW + g * 8
        pltpu.sync_copy(src_hbm.at[pl.ds(base, 8)], sidx)
        pltpu.sync_copy(dst_hbm.at[pl.ds(base, 8)], didx)
        gd = [
            pltpu.async_copy(ts_hbm.at[sidx.at[j]], rows.at[j], sem1)
            for j in range(8)
        ]
        for d in gd:
            d.wait()
        sd = [
            pltpu.async_copy(rows.at[j], acc.at[didx.at[j]], sem2,
                             add=True)
            for j in range(8)
        ]
        for d in sd:
            d.wait()

    plsc.subcore_barrier()
    pltpu.sync_copy(acc.at[pl.ds(s * STRIPE, STRIPE)],
                    out_hbm.at[c, pl.ds(s * STRIPE, STRIPE)])


@functools.partial(
    pl.kernel,
    out_type=(
        jax.ShapeDtypeStruct((PPAD, FW), _f32),
        jax.ShapeDtypeStruct((PPAD, FW), _f32),
    ),
    mesh=_sc_mesh,
    compiler_params=pltpu.CompilerParams(use_tc_tiling_on_sc=False),
    scratch_types=[
        pltpu.VMEM((8, 128), jnp.int32),
        pltpu.VMEM((8, 128), jnp.int32),
        pltpu.VMEM((8, 128, FW), _f32),
        pltpu.VMEM((8, 128, FW), _f32),
        pltpu.SemaphoreType.DMA,
        pltpu.SemaphoreType.DMA,
    ],
)
def _pair_kernel(n1_hbm, n2_hbm, u_hbm, v_hbm, a_out, b_out,
                 idx1, idx2, ar, br, sem1, sem2):
    c = lax.axis_index("c")
    s = lax.axis_index("s")
    w = s * NC + c

    @pl.loop(0, PCW // 8)
    def _grp(g):
        base = w * PCW + g * 8
        pltpu.sync_copy(n1_hbm.at[pl.ds(base, 8)], idx1)
        pltpu.sync_copy(n2_hbm.at[pl.ds(base, 8)], idx2)
        gd = [
            pltpu.async_copy(u_hbm.at[idx1.at[j]], ar.at[j], sem1)
            for j in range(8)
        ] + [
            pltpu.async_copy(v_hbm.at[idx2.at[j]], br.at[j], sem1)
            for j in range(8)
        ]
        for d in gd:
            d.wait()
        sd = [
            pltpu.async_copy(ar.at[j], a_out.at[pl.ds((base + j) * 128, 128)],
                             sem2)
            for j in range(8)
        ] + [
            pltpu.async_copy(br.at[j], b_out.at[pl.ds((base + j) * 128, 128)],
                             sem2)
            for j in range(8)
        ]
        for d in sd:
            d.wait()


def kernel(x, edge_index, bank, node1, node2, W1, b1, W2, b2, Wb, bb,
           Wf1, bf1, Wf2, bf2, Wf3, bf3):
    src = edge_index[0].astype(jnp.int32)
    dst = edge_index[1].astype(jnp.int32)
    n1 = node1.astype(jnp.int32)
    n2 = node2.astype(jnp.int32)

    Wc = Wf1 @ Wf2 @ Wf3
    bc = bf1 @ Wf2 @ Wf3 + bf2 @ Wf3 + bf3
    Wcc = jnp.concatenate([Wc[:50], Wc[50:]], axis=1)
    W_uv = W1 @ W2 @ Wcc
    cuv = (b1 @ W2) @ Wcc
    duv = b2 @ Wcc
    bank_w = Wb @ Wcc
    bank_b = bb @ Wcc

    srcp = jnp.pad(src, (0, EPAD - N_EDGES),
                   constant_values=DUMMY_NODE).reshape(ECH, 128)
    dstp = jnp.pad(dst, (0, EPAD - N_EDGES),
                   constant_values=DUMMY_NODE).reshape(ECH, 128)
    n1p = jnp.pad(n1, (0, PPAD - N_PAIRS),
                  constant_values=DUMMY_PAIR).reshape(PCH, 128)
    n2p = jnp.pad(n2, (0, PPAD - N_PAIRS),
                  constant_values=DUMMY_PAIR).reshape(PCH, 128)

    x32 = jnp.pad(x, ((0, NR - N_NODES), (0, 5)))
    w_pad = jnp.pad(W_uv, ((0, 5), (0, FW - 8)))
    ones_row = jnp.zeros((1, FW), _f32).at[0, 8].set(1.0)
    g = _node_matmul(x32, w_pad, ones_row, STRIPE)

    bank32 = jnp.pad(bank, ((0, 0), (0, 5)))
    bw_pad = jnp.pad(bank_w, ((0, 5), (0, FW - 8)))
    bank_bias = jnp.pad(bank_b.at[:4].add(bc), (0, FW - 8))[None, :]
    bank16 = _node_matmul(bank32, bw_pad, bank_bias, 5000)

    degp = _deg_kernel(dstp)
    d0 = degp[0].reshape(NR, 1)
    d1 = degp[1].reshape(NR, 1)
    ts1, dis = _row_kernel(_scale1_body, (FW, 1), (d0, d1, g))

    acc1 = _prop_kernel(srcp, dstp, ts1)
    out1, ts2 = _row_kernel(
        _scale2_body, (FW, FW), (acc1[0], acc1[1], g, dis))

    acc2 = _prop_kernel(srcp, dstp, ts2)
    cuv_row = jnp.pad(cuv, (0, FW - 8))[None, :]
    duv_row = jnp.pad(duv.at[:4].add(bc), (0, FW - 8))[None, :]
    (q,) = _row_kernel(
        _scale3_body, (FW,), (acc2[0], acc2[1], out1, dis, cuv_row, duv_row))

    ninf = jnp.full((NPF - N_NODES - N_BANK, 4), -jnp.inf, _f32)
    uf = jnp.concatenate([q[:N_NODES, 0:4], bank16[:, 0:4], ninf], axis=0)
    vf = jnp.concatenate([q[:N_NODES, 4:8], bank16[:, 4:8], ninf], axis=0)
    uf16 = jnp.pad(uf, ((0, 0), (0, FW - 4)))
    vf16 = jnp.pad(vf, ((0, 0), (0, FW - 4)))

    a16, b16 = _pair_kernel(n1p, n2p, uf16, vf16)
    a2d = a16.reshape(PPAD // 8, 128)
    b2d = b16.reshape(PPAD // 8, 128)
    out2d = _softmax_axis0(a2d, b2d, 1024)
    return out2d[:N_PAIRS // 8].reshape(N_PAIRS, FW)[:, :4]


# speedup vs baseline: 18.3243x; 1.0466x over previous
import functools

import jax
import jax.numpy as jnp
from jax import lax
from jax.experimental import pallas as pl
from jax.experimental.pallas import tpu as pltpu
from jax.experimental.pallas import tpu_sc as plsc

N_NODES = 100000
N_BANK = 10000
N_PAIRS = 500000
N_EDGES = 1600000
FW = 16
NC, NS = 2, 16
NW = NC * NS
NR = 100352
STRIPE = NR // NS
DUMMY_NODE = N_NODES
ECH = 12800
EPAD = ECH * 128
CW = ECH // NW
NPF = 110016
DUMMY_PAIR = N_NODES + N_BANK
PCH = 4096
PPAD = PCH * 128
PCW = PCH // NW

_sc_mesh = plsc.VectorSubcoreMesh(core_axis_name="c", subcore_axis_name="s")
_f32 = jnp.float32


def _matmul_body(x_ref, w_ref, c_ref, o_ref):
    o_ref[...] = (
        jnp.dot(x_ref[...], w_ref[...], preferred_element_type=_f32)
        + c_ref[...]
    )


def _node_matmul(x32, w_pad, row_const, bm):
    m = x32.shape[0]
    return pl.pallas_call(
        _matmul_body,
        grid=(m // bm,),
        in_specs=[
            pl.BlockSpec((bm, 32), lambda i: (i, 0)),
            pl.BlockSpec((32, FW), lambda i: (0, 0)),
            pl.BlockSpec((1, FW), lambda i: (0, 0)),
        ],
        out_specs=pl.BlockSpec((bm, FW), lambda i: (i, 0)),
        out_shape=jax.ShapeDtypeStruct((m, FW), _f32),
    )(x32, w_pad, row_const)


def _scale1_body(d0_ref, d1_ref, g_ref, ts_ref, dis_ref):
    dis = lax.rsqrt(d0_ref[...] + d1_ref[...] + 1.0)
    dis_ref[...] = dis
    ts_ref[...] = dis * g_ref[...]


def _scale2_body(a0_ref, a1_ref, g_ref, dis_ref, out1_ref, ts_ref):
    dis = dis_ref[...]
    out1 = dis * (a0_ref[...] + a1_ref[...]) + (dis * dis) * g_ref[...]
    out1_ref[...] = out1
    ts_ref[...] = dis * out1


def _scale3_body(a0_ref, a1_ref, out1_ref, dis_ref, cuv_ref, duv_ref, q_ref):
    dis = dis_ref[...]
    out1 = out1_ref[...]
    out2 = dis * (a0_ref[...] + a1_ref[...]) + (dis * dis) * out1
    r = out1[:, 8:9]
    q_ref[...] = out2 + r * cuv_ref[...] + duv_ref[...]


def _row_kernel(body, n_out, inputs, col_inputs=0, bm=STRIPE):
    m = inputs[0].shape[0]
    specs = []
    for a in inputs:
        if a.shape == (m, FW):
            specs.append(pl.BlockSpec((bm, FW), lambda i: (i, 0)))
        elif a.shape == (m, 1):
            specs.append(pl.BlockSpec((bm, 1), lambda i: (i, 0)))
        else:
            specs.append(pl.BlockSpec((1, FW), lambda i: (0, 0)))
    out_shapes = []
    out_specs = []
    for shp in n_out:
        out_shapes.append(jax.ShapeDtypeStruct((m, shp), _f32))
        out_specs.append(pl.BlockSpec((bm, shp), lambda i: (i, 0)))
    return pl.pallas_call(
        body,
        grid=(m // bm,),
        in_specs=specs,
        out_specs=out_specs,
        out_shape=out_shapes,
    )(*inputs)


def _softmax_body(a_ref, b_ref, o_ref, m_ref, s_ref):
    phase = pl.program_id(0)
    nb = pl.num_programs(1)
    i = pl.program_id(1)

    @pl.when(jnp.logical_and(phase == 0, i == 0))
    def _init():
        m_ref[...] = jnp.full_like(m_ref, -jnp.inf)
        s_ref[...] = jnp.zeros_like(s_ref)

    @pl.when(phase == 0)
    def _acc():
        blk = a_ref[...] + b_ref[...]
        bm_ = jnp.max(blk, axis=0, keepdims=True)
        m_old = m_ref[...]
        m_new = jnp.maximum(m_old, bm_)
        bs = jnp.sum(jnp.exp(blk - m_new), axis=0, keepdims=True)
        s_ref[...] = s_ref[...] * jnp.exp(m_old - m_new) + bs
        m_ref[...] = m_new

        @pl.when(i == nb - 1)
        def _lanefold():
            mm = m_ref[...]
            ss = s_ref[...]
            for k in (16, 32, 64):
                mr = pltpu.roll(mm, k, 1)
                sr = pltpu.roll(ss, k, 1)
                mn = jnp.maximum(mm, mr)
                ss = ss * jnp.exp(mm - mn) + sr * jnp.exp(mr - mn)
                mm = mn
            m_ref[...] = mm
            s_ref[...] = ss

    @pl.when(phase == 1)
    def _emit():
        o_ref[...] = jnp.exp(a_ref[...] + b_ref[...] - m_ref[...]) * (1.0 / s_ref[...])


def _softmax_axis0(a2d, b2d, bm):
    r = a2d.shape[0]
    spec = pl.BlockSpec((bm, 128), lambda p, i: (i, 0))
    return pl.pallas_call(
        _softmax_body,
        grid=(2, r // bm),
        in_specs=[spec, spec],
        out_specs=pl.BlockSpec((bm, 128), lambda p, i: (i, 0)),
        out_shape=jax.ShapeDtypeStruct((r, 128), _f32),
        scratch_shapes=[
            pltpu.VMEM((1, 128), _f32),
            pltpu.VMEM((1, 128), _f32),
        ],
        compiler_params=pltpu.CompilerParams(
            dimension_semantics=("arbitrary", "arbitrary")
        ),
    )(a2d, b2d)


@functools.partial(
    pl.kernel,
    out_type=jax.ShapeDtypeStruct((NC, NR), _f32),
    mesh=_sc_mesh,
    compiler_params=pltpu.CompilerParams(use_tc_tiling_on_sc=False),
    scratch_types=[
        pltpu.VMEM((CW, 128), jnp.int32),
        pltpu.VMEM((128,), _f32),
        pltpu.VMEM((STRIPE,), _f32),
        pltpu.VMEM_SHARED((NR,), _f32),
        pltpu.SemaphoreType.DMA,
    ],
)
def _deg_kernel(dst_hbm, out_hbm, didx, ones_v, zer_v, acc, sem):
    c = lax.axis_index("c")
    s = lax.axis_index("s")
    w = s * NC + c

    @pl.loop(0, STRIPE // 16)
    def _zf(i):
        zer_v[pl.ds(i * 16, 16)] = jnp.zeros((16,), _f32)

    for i in range(8):
        ones_v[pl.ds(i * 16, 16)] = jnp.ones((16,), _f32)

    pltpu.sync_copy(zer_v, acc.at[pl.ds(s * STRIPE, STRIPE)])
    pltpu.sync_copy(dst_hbm.at[pl.ds(w * CW, CW)], didx)
    plsc.subcore_barrier()

    @pl.loop(0, CW // 16)
    def _grp(g):
        descs = [
            pltpu.async_copy(ones_v, acc.at[didx.at[g * 16 + j]], sem,
                             add=True)
            for j in range(16)
        ]
        for d in descs:
            d.wait()

    plsc.subcore_barrier()
    pltpu.sync_copy(acc.at[pl.ds(s * STRIPE, STRIPE)],
                    out_hbm.at[c, pl.ds(s * STRIPE, STRIPE)])


@functools.partial(
    pl.kernel,
    out_type=jax.ShapeDtypeStruct((NC, NR, FW), _f32),
    mesh=_sc_mesh,
    compiler_params=pltpu.CompilerParams(use_tc_tiling_on_sc=False),
    scratch_types=[
        pltpu.VMEM((16, 2, 128), jnp.int32),
        pltpu.VMEM((8, 128, FW), _f32),
        pltpu.VMEM((128, FW), _f32),
        pltpu.VMEM_SHARED((NR, FW), _f32),
        pltpu.SemaphoreType.DMA,
        pltpu.SemaphoreType.DMA,
    ],
)
def _prop_kernel(eidx_hbm, ts_hbm, out_hbm,
                 eidx, rows, zer, acc, sem1, sem2):
    c = lax.axis_index("c")
    s = lax.axis_index("s")
    w = s * NC + c

    @pl.loop(0, 128)
    def _zf(i):
        zer[i, :] = jnp.zeros((16,), _f32)

    @pl.loop(0, STRIPE // 128)
    def _zs(k):
        pltpu.sync_copy(zer, acc.at[pl.ds(s * STRIPE + k * 128, 128)])

    plsc.subcore_barrier()

    @pl.loop(0, CW // 16)
    def _sup(g):
        base = w * CW + g * 16
        pltpu.sync_copy(eidx_hbm.at[pl.ds(base, 16)], eidx)
        for h in range(2):
            gd = [
                pltpu.async_copy(ts_hbm.at[eidx.at[h * 8 + j, 0]],
                                 rows.at[j], sem1)
                for j in range(8)
            ]
            for d in gd:
                d.wait()
            sd = [
                pltpu.async_copy(rows.at[j], acc.at[eidx.at[h * 8 + j, 1]],
                                 sem2, add=True)
                for j in range(8)
            ]
            for d in sd:
                d.wait()

    plsc.subcore_barrier()
    pltpu.sync_copy(acc.at[pl.ds(s * STRIPE, STRIPE)],
                    out_hbm.at[c, pl.ds(s * STRIPE, STRIPE)])


@functools.partial(
    pl.kernel,
    out_type=(
        jax.ShapeDtypeStruct((PPAD, FW), _f32),
        jax.ShapeDtypeStruct((PPAD, FW), _f32),
    ),
    mesh=_sc_mesh,
    compiler_params=pltpu.CompilerParams(use_tc_tiling_on_sc=False),
    scratch_types=[
        pltpu.VMEM((PCW, 128), jnp.int32),
        pltpu.VMEM((PCW, 128), jnp.int32),
        pltpu.VMEM((8, 128, FW), _f32),
        pltpu.VMEM((8, 128, FW), _f32),
        pltpu.SemaphoreType.DMA,
        pltpu.SemaphoreType.DMA,
    ],
)
def _pair_kernel(n1_hbm, n2_hbm, u_hbm, v_hbm, a_out, b_out,
                 idx1, idx2, ar, br, sem1, sem2):
    c = lax.axis_index("c")
    s = lax.axis_index("s")
    w = s * NC + c
    pltpu.sync_copy(n1_hbm.at[pl.ds(w * PCW, PCW)], idx1)
    pltpu.sync_copy(n2_hbm.at[pl.ds(w * PCW, PCW)], idx2)

    @pl.loop(0, PCW // 8)
    def _grp(g):
        base = w * PCW + g * 8
        gd = [
            pltpu.async_copy(u_hbm.at[idx1.at[g * 8 + j]], ar.at[j], sem1)
            for j in range(8)
        ] + [
            pltpu.async_copy(v_hbm.at[idx2.at[g * 8 + j]], br.at[j], sem1)
            for j in range(8)
        ]
        for d in gd:
            d.wait()
        sd = [
            pltpu.async_copy(ar.at[j], a_out.at[pl.ds((base + j) * 128, 128)],
                             sem2)
            for j in range(8)
        ] + [
            pltpu.async_copy(br.at[j], b_out.at[pl.ds((base + j) * 128, 128)],
                             sem2)
            for j in range(8)
        ]
        for d in sd:
            d.wait()


def kernel(x, edge_index, bank, node1, node2, W1, b1, W2, b2, Wb, bb,
           Wf1, bf1, Wf2, bf2, Wf3, bf3):
    src = edge_index[0].astype(jnp.int32)
    dst = edge_index[1].astype(jnp.int32)
    n1 = node1.astype(jnp.int32)
    n2 = node2.astype(jnp.int32)

    Wc = Wf1 @ Wf2 @ Wf3
    bc = bf1 @ Wf2 @ Wf3 + bf2 @ Wf3 + bf3
    Wcc = jnp.concatenate([Wc[:50], Wc[50:]], axis=1)
    W_uv = W1 @ W2 @ Wcc
    cuv = (b1 @ W2) @ Wcc
    duv = b2 @ Wcc
    bank_w = Wb @ Wcc
    bank_b = bb @ Wcc

    srcp = jnp.pad(src, (0, EPAD - N_EDGES),
                   constant_values=DUMMY_NODE).reshape(ECH, 128)
    dstp = jnp.pad(dst, (0, EPAD - N_EDGES),
                   constant_values=DUMMY_NODE).reshape(ECH, 128)
    n1p = jnp.pad(n1, (0, PPAD - N_PAIRS),
                  constant_values=DUMMY_PAIR).reshape(PCH, 128)
    n2p = jnp.pad(n2, (0, PPAD - N_PAIRS),
                  constant_values=DUMMY_PAIR).reshape(PCH, 128)

    x32 = jnp.pad(x, ((0, NR - N_NODES), (0, 5)))
    w_pad = jnp.pad(W_uv, ((0, 5), (0, FW - 8)))
    ones_row = jnp.zeros((1, FW), _f32).at[0, 8].set(1.0)
    g = _node_matmul(x32, w_pad, ones_row, STRIPE)

    bank32 = jnp.pad(bank, ((0, 0), (0, 5)))
    bw_pad = jnp.pad(bank_w, ((0, 5), (0, FW - 8)))
    bank_bias = jnp.pad(bank_b.at[:4].add(bc), (0, FW - 8))[None, :]
    bank16 = _node_matmul(bank32, bw_pad, bank_bias, 5000)

    degp = _deg_kernel(dstp)
    d0 = degp[0].reshape(NR, 1)
    d1 = degp[1].reshape(NR, 1)
    ts1, dis = _row_kernel(_scale1_body, (FW, 1), (d0, d1, g))

    eidxp = jnp.stack([srcp, dstp], axis=1)
    acc1 = _prop_kernel(eidxp, ts1)
    out1, ts2 = _row_kernel(
        _scale2_body, (FW, FW), (acc1[0], acc1[1], g, dis))

    acc2 = _prop_kernel(eidxp, ts2)
    cuv_row = jnp.pad(cuv, (0, FW - 8))[None, :]
    duv_row = jnp.pad(duv.at[:4].add(bc), (0, FW - 8))[None, :]
    (q,) = _row_kernel(
        _scale3_body, (FW,), (acc2[0], acc2[1], out1, dis, cuv_row, duv_row))

    ninf = jnp.full((NPF - N_NODES - N_BANK, 4), -jnp.inf, _f32)
    uf = jnp.concatenate([q[:N_NODES, 0:4], bank16[:, 0:4], ninf], axis=0)
    vf = jnp.concatenate([q[:N_NODES, 4:8], bank16[:, 4:8], ninf], axis=0)
    uf16 = jnp.pad(uf, ((0, 0), (0, FW - 4)))
    vf16 = jnp.pad(vf, ((0, 0), (0, FW - 4)))

    a16, b16 = _pair_kernel(n1p, n2p, uf16, vf16)
    a2d = a16.reshape(PPAD // 8, 128)
    b2d = b16.reshape(PPAD // 8, 128)
    out2d = _softmax_axis0(a2d, b2d, 1024)
    return out2d[:N_PAIRS // 8].reshape(N_PAIRS, FW)[:, :4]


# speedup vs baseline: 18.4672x; 1.0078x over previous
import functools

import jax
import jax.numpy as jnp
from jax import lax
from jax.experimental import pallas as pl
from jax.experimental.pallas import tpu as pltpu
from jax.experimental.pallas import tpu_sc as plsc

N_NODES = 100000
N_BANK = 10000
N_PAIRS = 500000
N_EDGES = 1600000
FW = 16
NC, NS = 2, 16
NW = NC * NS
NR = 100352
STRIPE = NR // NS
DUMMY_NODE = N_NODES
ECH = 12800
EPAD = ECH * 128
CW = ECH // NW
NPF = 110016
DUMMY_PAIR = N_NODES + N_BANK
PCH = 4096
PPAD = PCH * 128
PCW = PCH // NW

_sc_mesh = plsc.VectorSubcoreMesh(core_axis_name="c", subcore_axis_name="s")
_f32 = jnp.float32


def _matmul_body(x_ref, w_ref, c_ref, o_ref):
    o_ref[...] = (
        jnp.dot(x_ref[...], w_ref[...], preferred_element_type=_f32)
        + c_ref[...]
    )


def _node_matmul(x32, w_pad, row_const, bm):
    m = x32.shape[0]
    return pl.pallas_call(
        _matmul_body,
        grid=(m // bm,),
        in_specs=[
            pl.BlockSpec((bm, 32), lambda i: (i, 0)),
            pl.BlockSpec((32, FW), lambda i: (0, 0)),
            pl.BlockSpec((1, FW), lambda i: (0, 0)),
        ],
        out_specs=pl.BlockSpec((bm, FW), lambda i: (i, 0)),
        out_shape=jax.ShapeDtypeStruct((m, FW), _f32),
    )(x32, w_pad, row_const)


def _scale1_body(d0_ref, d1_ref, g_ref, ts_ref, dis_ref):
    dis = lax.rsqrt(d0_ref[...] + d1_ref[...] + 1.0)
    dis_ref[...] = dis
    ts_ref[...] = dis * g_ref[...]


def _scale2_body(a0_ref, a1_ref, g_ref, dis_ref, out1_ref, ts_ref):
    dis = dis_ref[...]
    out1 = dis * (a0_ref[...] + a1_ref[...]) + (dis * dis) * g_ref[...]
    out1_ref[...] = out1
    ts_ref[...] = dis * out1


def _scale3_body(a0_ref, a1_ref, out1_ref, dis_ref, cuv_ref, duv_ref, q_ref):
    dis = dis_ref[...]
    out1 = out1_ref[...]
    out2 = dis * (a0_ref[...] + a1_ref[...]) + (dis * dis) * out1
    r = out1[:, 8:9]
    q_ref[...] = out2 + r * cuv_ref[...] + duv_ref[...]


def _row_kernel(body, n_out, inputs, col_inputs=0, bm=STRIPE):
    m = inputs[0].shape[0]
    specs = []
    for a in inputs:
        if a.shape == (m, FW):
            specs.append(pl.BlockSpec((bm, FW), lambda i: (i, 0)))
        elif a.shape == (m, 1):
            specs.append(pl.BlockSpec((bm, 1), lambda i: (i, 0)))
        else:
            specs.append(pl.BlockSpec((1, FW), lambda i: (0, 0)))
    out_shapes = []
    out_specs = []
    for shp in n_out:
        out_shapes.append(jax.ShapeDtypeStruct((m, shp), _f32))
        out_specs.append(pl.BlockSpec((bm, shp), lambda i: (i, 0)))
    return pl.pallas_call(
        body,
        grid=(m // bm,),
        in_specs=specs,
        out_specs=out_specs,
        out_shape=out_shapes,
    )(*inputs)


def _softmax_body(a_ref, b_ref, o_ref, m_ref, s_ref):
    phase = pl.program_id(0)
    nb = pl.num_programs(1)
    i = pl.program_id(1)

    @pl.when(jnp.logical_and(phase == 0, i == 0))
    def _init():
        m_ref[...] = jnp.full_like(m_ref, -jnp.inf)
        s_ref[...] = jnp.zeros_like(s_ref)

    @pl.when(phase == 0)
    def _acc():
        blk = a_ref[...] + b_ref[...]
        bm_ = jnp.max(blk, axis=0, keepdims=True)
        m_old = m_ref[...]
        m_new = jnp.maximum(m_old, bm_)
        bs = jnp.sum(jnp.exp(blk - m_new), axis=0, keepdims=True)
        s_ref[...] = s_ref[...] * jnp.exp(m_old - m_new) + bs
        m_ref[...] = m_new

        @pl.when(i == nb - 1)
        def _lanefold():
            mm = m_ref[...]
            ss = s_ref[...]
            for k in (16, 32, 64):
                mr = pltpu.roll(mm, k, 1)
                sr = pltpu.roll(ss, k, 1)
                mn = jnp.maximum(mm, mr)
                ss = ss * jnp.exp(mm - mn) + sr * jnp.exp(mr - mn)
                mm = mn
            m_ref[...] = mm
            s_ref[...] = ss

    @pl.when(phase == 1)
    def _emit():
        o_ref[...] = jnp.exp(a_ref[...] + b_ref[...] - m_ref[...]) * (1.0 / s_ref[...])


def _softmax_axis0(a2d, b2d, bm):
    r = a2d.shape[0]
    spec = pl.BlockSpec((bm, 128), lambda p, i: (i, 0))
    return pl.pallas_call(
        _softmax_body,
        grid=(2, r // bm),
        in_specs=[spec, spec],
        out_specs=pl.BlockSpec((bm, 128), lambda p, i: (i, 0)),
        out_shape=jax.ShapeDtypeStruct((r, 128), _f32),
        scratch_shapes=[
            pltpu.VMEM((1, 128), _f32),
            pltpu.VMEM((1, 128), _f32),
        ],
        compiler_params=pltpu.CompilerParams(
            dimension_semantics=("arbitrary", "arbitrary")
        ),
    )(a2d, b2d)


@functools.partial(
    pl.kernel,
    out_type=jax.ShapeDtypeStruct((NC, NR), _f32),
    mesh=_sc_mesh,
    compiler_params=pltpu.CompilerParams(use_tc_tiling_on_sc=False),
    scratch_types=[
        pltpu.VMEM((CW, 128), jnp.int32),
        pltpu.VMEM((128,), _f32),
        pltpu.VMEM((STRIPE,), _f32),
        pltpu.VMEM_SHARED((NR,), _f32),
        pltpu.SemaphoreType.DMA,
    ],
)
def _deg_kernel(dst_hbm, out_hbm, didx, ones_v, zer_v, acc, sem):
    c = lax.axis_index("c")
    s = lax.axis_index("s")
    w = s * NC + c

    @pl.loop(0, STRIPE // 16)
    def _zf(i):
        zer_v[pl.ds(i * 16, 16)] = jnp.zeros((16,), _f32)

    for i in range(8):
        ones_v[pl.ds(i * 16, 16)] = jnp.ones((16,), _f32)

    pltpu.sync_copy(zer_v, acc.at[pl.ds(s * STRIPE, STRIPE)])
    pltpu.sync_copy(dst_hbm.at[pl.ds(w * CW, CW)], didx)
    plsc.subcore_barrier()

    @pl.loop(0, CW // 16)
    def _grp(g):
        descs = [
            pltpu.async_copy(ones_v, acc.at[didx.at[g * 16 + j]], sem,
                             add=True)
            for j in range(16)
        ]
        for d in descs:
            d.wait()

    plsc.subcore_barrier()
    pltpu.sync_copy(acc.at[pl.ds(s * STRIPE, STRIPE)],
                    out_hbm.at[c, pl.ds(s * STRIPE, STRIPE)])


@functools.partial(
    pl.kernel,
    out_type=jax.ShapeDtypeStruct((NC, NR, FW), _f32),
    mesh=_sc_mesh,
    compiler_params=pltpu.CompilerParams(use_tc_tiling_on_sc=False),
    scratch_types=[
        pltpu.VMEM((16, 128), jnp.int32),
        pltpu.VMEM((16, 128), jnp.int32),
        pltpu.VMEM((8, 128, FW), _f32),
        pltpu.VMEM((128, FW), _f32),
        pltpu.VMEM_SHARED((NR, FW), _f32),
        pltpu.SemaphoreType.DMA,
        pltpu.SemaphoreType.DMA,
    ],
)
def _prop_kernel(src_hbm, dst_hbm, ts_hbm, out_hbm,
                 sidx, didx, rows, zer, acc, sem1, sem2):
    c = lax.axis_index("c")
    s = lax.axis_index("s")
    w = s * NC + c

    @pl.loop(0, 128)
    def _zf(i):
        zer[i, :] = jnp.zeros((16,), _f32)

    @pl.loop(0, STRIPE // 128)
    def _zs(k):
        pltpu.sync_copy(zer, acc.at[pl.ds(s * STRIPE + k * 128, 128)])

    plsc.subcore_barrier()

    @pl.loop(0, CW // 16)
    def _sup(g):
        base = w * CW + g * 16
        pltpu.sync_copy(src_hbm.at[pl.ds(base, 16)], sidx)
        pltpu.sync_copy(dst_hbm.at[pl.ds(base, 16)], didx)
        for h in range(2):
            gd = [
                pltpu.async_copy(ts_hbm.at[sidx.at[h * 8 + j]],
                                 rows.at[j], sem1)
                for j in range(8)
            ]
            sd = []
            for j in range(8):
                gd[j].wait()
                sd.append(
                    pltpu.async_copy(rows.at[j], acc.at[didx.at[h * 8 + j]],
                                     sem2, add=True))
            for d in sd:
                d.wait()

    plsc.subcore_barrier()
    pltpu.sync_copy(acc.at[pl.ds(s * STRIPE, STRIPE)],
                    out_hbm.at[c, pl.ds(s * STRIPE, STRIPE)])


@functools.partial(
    pl.kernel,
    out_type=(
        jax.ShapeDtypeStruct((PPAD, FW), _f32),
        jax.ShapeDtypeStruct((PPAD, FW), _f32),
    ),
    mesh=_sc_mesh,
    compiler_params=pltpu.CompilerParams(use_tc_tiling_on_sc=False),
    scratch_types=[
        pltpu.VMEM((PCW, 128), jnp.int32),
        pltpu.VMEM((PCW, 128), jnp.int32),
        pltpu.VMEM((8, 128, FW), _f32),
        pltpu.VMEM((8, 128, FW), _f32),
        pltpu.SemaphoreType.DMA,
        pltpu.SemaphoreType.DMA,
    ],
)
def _pair_kernel(n1_hbm, n2_hbm, u_hbm, v_hbm, a_out, b_out,
                 idx1, idx2, ar, br, sem1, sem2):
    c = lax.axis_index("c")
    s = lax.axis_index("s")
    w = s * NC + c
    pltpu.sync_copy(n1_hbm.at[pl.ds(w * PCW, PCW)], idx1)
    pltpu.sync_copy(n2_hbm.at[pl.ds(w * PCW, PCW)], idx2)

    @pl.loop(0, PCW // 8)
    def _grp(g):
        base = w * PCW + g * 8
        ga = [
            pltpu.async_copy(u_hbm.at[idx1.at[g * 8 + j]], ar.at[j], sem1)
            for j in range(8)
        ]
        gb = [
            pltpu.async_copy(v_hbm.at[idx2.at[g * 8 + j]], br.at[j], sem1)
            for j in range(8)
        ]
        sd = []
        for j in range(8):
            ga[j].wait()
            sd.append(pltpu.async_copy(
                ar.at[j], a_out.at[pl.ds((base + j) * 128, 128)], sem2))
        for j in range(8):
            gb[j].wait()
            sd.append(pltpu.async_copy(
                br.at[j], b_out.at[pl.ds((base + j) * 128, 128)], sem2))
        for d in sd:
            d.wait()


def kernel(x, edge_index, bank, node1, node2, W1, b1, W2, b2, Wb, bb,
           Wf1, bf1, Wf2, bf2, Wf3, bf3):
    src = edge_index[0].astype(jnp.int32)
    dst = edge_index[1].astype(jnp.int32)
    n1 = node1.astype(jnp.int32)
    n2 = node2.astype(jnp.int32)

    Wc = Wf1 @ Wf2 @ Wf3
    bc = bf1 @ Wf2 @ Wf3 + bf2 @ Wf3 + bf3
    Wcc = jnp.concatenate([Wc[:50], Wc[50:]], axis=1)
    W_uv = W1 @ W2 @ Wcc
    cuv = (b1 @ W2) @ Wcc
    duv = b2 @ Wcc
    bank_w = Wb @ Wcc
    bank_b = bb @ Wcc

    srcp = jnp.pad(src, (0, EPAD - N_EDGES),
                   constant_values=DUMMY_NODE).reshape(ECH, 128)
    dstp = jnp.pad(dst, (0, EPAD - N_EDGES),
                   constant_values=DUMMY_NODE).reshape(ECH, 128)
    n1p = jnp.pad(n1, (0, PPAD - N_PAIRS),
                  constant_values=DUMMY_PAIR).reshape(PCH, 128)
    n2p = jnp.pad(n2, (0, PPAD - N_PAIRS),
                  constant_values=DUMMY_PAIR).reshape(PCH, 128)

    x32 = jnp.pad(x, ((0, NR - N_NODES), (0, 5)))
    w_pad = jnp.pad(W_uv, ((0, 5), (0, FW - 8)))
    ones_row = jnp.zeros((1, FW), _f32).at[0, 8].set(1.0)
    g = _node_matmul(x32, w_pad, ones_row, STRIPE)

    bank32 = jnp.pad(bank, ((0, 0), (0, 5)))
    bw_pad = jnp.pad(bank_w, ((0, 5), (0, FW - 8)))
    bank_bias = jnp.pad(bank_b.at[:4].add(bc), (0, FW - 8))[None, :]
    bank16 = _node_matmul(bank32, bw_pad, bank_bias, 5000)

    degp = _deg_kernel(dstp)
    d0 = degp[0].reshape(NR, 1)
    d1 = degp[1].reshape(NR, 1)
    ts1, dis = _row_kernel(_scale1_body, (FW, 1), (d0, d1, g))

    acc1 = _prop_kernel(srcp, dstp, ts1)
    out1, ts2 = _row_kernel(
        _scale2_body, (FW, FW), (acc1[0], acc1[1], g, dis))

    acc2 = _prop_kernel(srcp, dstp, ts2)
    cuv_row = jnp.pad(cuv, (0, FW - 8))[None, :]
    duv_row = jnp.pad(duv.at[:4].add(bc), (0, FW - 8))[None, :]
    (q,) = _row_kernel(
        _scale3_body, (FW,), (acc2[0], acc2[1], out1, dis, cuv_row, duv_row))

    ninf = jnp.full((NPF - N_NODES - N_BANK, 4), -jnp.inf, _f32)
    uf = jnp.concatenate([q[:N_NODES, 0:4], bank16[:, 0:4], ninf], axis=0)
    vf = jnp.concatenate([q[:N_NODES, 4:8], bank16[:, 4:8], ninf], axis=0)
    uf16 = jnp.pad(uf, ((0, 0), (0, FW - 4)))
    vf16 = jnp.pad(vf, ((0, 0), (0, FW - 4)))

    a16, b16 = _pair_kernel(n1p, n2p, uf16, vf16)
    a2d = a16.reshape(PPAD // 8, 128)
    b2d = b16.reshape(PPAD // 8, 128)
    out2d = _softmax_axis0(a2d, b2d, 1024)
    return out2d[:N_PAIRS // 8].reshape(N_PAIRS, FW)[:, :4]


# speedup vs baseline: 21.1092x; 1.1431x over previous
import functools

import jax
import jax.numpy as jnp
from jax import lax
from jax.experimental import pallas as pl
from jax.experimental.pallas import tpu as pltpu
from jax.experimental.pallas import tpu_sc as plsc

N_NODES = 100000
N_BANK = 10000
N_PAIRS = 500000
N_EDGES = 1600000
FW = 16
NC, NS = 2, 16
NW = NC * NS
NR = 100352
STRIPE = NR // NS
DUMMY_NODE = N_NODES
ECH = 12800
EPAD = ECH * 128
CW = ECH // NW
NPF2 = 112896
DUMMY_PAIR = N_NODES + N_BANK
PCH = 4096
PPAD = PCH * 128
PCW = PCH // NW

_sc_mesh = plsc.VectorSubcoreMesh(core_axis_name="c", subcore_axis_name="s")
_f32 = jnp.float32


def _matmul_body(x_ref, w_ref, c_ref, o_ref):
    o_ref[...] = (
        jnp.dot(x_ref[...], w_ref[...], preferred_element_type=_f32)
        + c_ref[...]
    )


def _node_matmul(x32, w_pad, row_const, bm):
    m = x32.shape[0]
    return pl.pallas_call(
        _matmul_body,
        grid=(m // bm,),
        in_specs=[
            pl.BlockSpec((bm, 32), lambda i: (i, 0)),
            pl.BlockSpec((32, FW), lambda i: (0, 0)),
            pl.BlockSpec((1, FW), lambda i: (0, 0)),
        ],
        out_specs=pl.BlockSpec((bm, FW), lambda i: (i, 0)),
        out_shape=jax.ShapeDtypeStruct((m, FW), _f32),
    )(x32, w_pad, row_const)


def _scale1_body(d0_ref, d1_ref, g_ref, ts_ref, dis_ref):
    dis = lax.rsqrt(d0_ref[...] + d1_ref[...] + 1.0)
    dis_ref[...] = dis
    ts_ref[...] = dis * g_ref[...]


def _scale2_body(a0_ref, a1_ref, g_ref, dis_ref, out1_ref, ts_ref):
    dis = dis_ref[...]
    out1 = dis * (a0_ref[...] + a1_ref[...]) + (dis * dis) * g_ref[...]
    out1_ref[...] = out1
    ts_ref[...] = dis * out1


def _scale3_body(a0_ref, a1_ref, out1_ref, dis_ref, cuv_ref, duv_ref,
                 bank_ref, q_ref):
    i = pl.program_id(0)

    @pl.when(i < 16)
    def _node():
        dis = dis_ref[...]
        out1 = out1_ref[...]
        out2 = dis * (a0_ref[...] + a1_ref[...]) + (dis * dis) * out1
        r = out1[:, 8:9]
        q_ref[...] = out2 + r * cuv_ref[...] + duv_ref[...]

    @pl.when(i >= 16)
    def _bank():
        q_ref[...] = bank_ref[...]


def _scale3_kernel(a0, a1, out1, dis, cuv_row, duv_row, bank_tab):
    bm = STRIPE
    row16 = pl.BlockSpec((bm, FW), lambda i: (jnp.minimum(i, 15), 0))
    return pl.pallas_call(
        _scale3_body,
        grid=(18,),
        in_specs=[
            row16, row16,
            row16,
            pl.BlockSpec((bm, 1), lambda i: (jnp.minimum(i, 15), 0)),
            pl.BlockSpec((1, FW), lambda i: (0, 0)),
            pl.BlockSpec((1, FW), lambda i: (0, 0)),
            pl.BlockSpec((bm, FW), lambda i: (jnp.maximum(i - 16, 0), 0)),
        ],
        out_specs=pl.BlockSpec((bm, FW), lambda i: (i, 0)),
        out_shape=jax.ShapeDtypeStruct((NPF2, FW), _f32),
    )(a0, a1, out1, dis, cuv_row, duv_row, bank_tab)


def _row_kernel(body, n_out, inputs, col_inputs=0, bm=STRIPE):
    m = inputs[0].shape[0]
    specs = []
    for a in inputs:
        if a.shape == (m, FW):
            specs.append(pl.BlockSpec((bm, FW), lambda i: (i, 0)))
        elif a.shape == (m, 1):
            specs.append(pl.BlockSpec((bm, 1), lambda i: (i, 0)))
        else:
            specs.append(pl.BlockSpec((1, FW), lambda i: (0, 0)))
    out_shapes = []
    out_specs = []
    for shp in n_out:
        out_shapes.append(jax.ShapeDtypeStruct((m, shp), _f32))
        out_specs.append(pl.BlockSpec((bm, shp), lambda i: (i, 0)))
    return pl.pallas_call(
        body,
        grid=(m // bm,),
        in_specs=specs,
        out_specs=out_specs,
        out_shape=out_shapes,
    )(*inputs)


def _softmax_body(a_ref, b_ref, o_ref, m_ref, s_ref):
    phase = pl.program_id(0)
    nb = pl.num_programs(1)
    i = pl.program_id(1)

    @pl.when(jnp.logical_and(phase == 0, i == 0))
    def _init():
        m_ref[...] = jnp.full_like(m_ref, -jnp.inf)
        s_ref[...] = jnp.zeros_like(s_ref)

    @pl.when(phase == 0)
    def _acc():
        blk = a_ref[...] + pltpu.roll(b_ref[...], 124, 1)
        bm_ = jnp.max(blk, axis=0, keepdims=True)
        m_old = m_ref[...]
        m_new = jnp.maximum(m_old, bm_)
        bs = jnp.sum(jnp.exp(blk - m_new), axis=0, keepdims=True)
        s_ref[...] = s_ref[...] * jnp.exp(m_old - m_new) + bs
        m_ref[...] = m_new

        @pl.when(i == nb - 1)
        def _lanefold():
            mm = m_ref[...]
            ss = s_ref[...]
            for k in (16, 32, 64):
                mr = pltpu.roll(mm, k, 1)
                sr = pltpu.roll(ss, k, 1)
                mn = jnp.maximum(mm, mr)
                ss = ss * jnp.exp(mm - mn) + sr * jnp.exp(mr - mn)
                mm = mn
            m_ref[...] = mm
            s_ref[...] = ss

    @pl.when(phase == 1)
    def _emit():
        o_ref[...] = (jnp.exp(a_ref[...] + pltpu.roll(b_ref[...], 124, 1)
                              - m_ref[...]) * (1.0 / s_ref[...]))


def _softmax_axis0(a2d, b2d, bm):
    r = a2d.shape[0]
    spec = pl.BlockSpec((bm, 128), lambda p, i: (i, 0))
    return pl.pallas_call(
        _softmax_body,
        grid=(2, r // bm),
        in_specs=[spec, spec],
        out_specs=pl.BlockSpec((bm, 128), lambda p, i: (i, 0)),
        out_shape=jax.ShapeDtypeStruct((r, 128), _f32),
        scratch_shapes=[
            pltpu.VMEM((1, 128), _f32),
            pltpu.VMEM((1, 128), _f32),
        ],
        compiler_params=pltpu.CompilerParams(
            dimension_semantics=("arbitrary", "arbitrary")
        ),
    )(a2d, b2d)


@functools.partial(
    pl.kernel,
    out_type=jax.ShapeDtypeStruct((NC, NR), _f32),
    mesh=_sc_mesh,
    compiler_params=pltpu.CompilerParams(use_tc_tiling_on_sc=False),
    scratch_types=[
        pltpu.VMEM((CW, 128), jnp.int32),
        pltpu.VMEM((128,), _f32),
        pltpu.VMEM((STRIPE,), _f32),
        pltpu.VMEM_SHARED((NR,), _f32),
        pltpu.SemaphoreType.DMA,
    ],
)
def _deg_kernel(dst_hbm, out_hbm, didx, ones_v, zer_v, acc, sem):
    c = lax.axis_index("c")
    s = lax.axis_index("s")
    w = s * NC + c

    @pl.loop(0, STRIPE // 16)
    def _zf(i):
        zer_v[pl.ds(i * 16, 16)] = jnp.zeros((16,), _f32)

    for i in range(8):
        ones_v[pl.ds(i * 16, 16)] = jnp.ones((16,), _f32)

    pltpu.sync_copy(zer_v, acc.at[pl.ds(s * STRIPE, STRIPE)])
    pltpu.sync_copy(dst_hbm.at[pl.ds(w * CW, CW)], didx)
    plsc.subcore_barrier()

    @pl.loop(0, CW // 16)
    def _grp(g):
        descs = [
            pltpu.async_copy(ones_v, acc.at[didx.at[g * 16 + j]], sem,
                             add=True)
            for j in range(16)
        ]
        for d in descs:
            d.wait()

    plsc.subcore_barrier()
    pltpu.sync_copy(acc.at[pl.ds(s * STRIPE, STRIPE)],
                    out_hbm.at[c, pl.ds(s * STRIPE, STRIPE)])


@functools.partial(
    pl.kernel,
    out_type=jax.ShapeDtypeStruct((NC, NR, FW), _f32),
    mesh=_sc_mesh,
    compiler_params=pltpu.CompilerParams(use_tc_tiling_on_sc=False),
    scratch_types=[
        pltpu.VMEM((16, 128), jnp.int32),
        pltpu.VMEM((16, 128), jnp.int32),
        pltpu.VMEM((8, 128, FW), _f32),
        pltpu.VMEM((128, FW), _f32),
        pltpu.VMEM_SHARED((NR, FW), _f32),
        pltpu.SemaphoreType.DMA,
        pltpu.SemaphoreType.DMA,
    ],
)
def _prop_kernel(src_hbm, dst_hbm, ts_hbm, out_hbm,
                 sidx, didx, rows, zer, acc, sem1, sem2):
    c = lax.axis_index("c")
    s = lax.axis_index("s")
    w = s * NC + c

    @pl.loop(0, 128)
    def _zf(i):
        zer[i, :] = jnp.zeros((16,), _f32)

    @pl.loop(0, STRIPE // 128)
    def _zs(k):
        pltpu.sync_copy(zer, acc.at[pl.ds(s * STRIPE + k * 128, 128)])

    plsc.subcore_barrier()

    @pl.loop(0, CW // 16)
    def _sup(g):
        base = w * CW + g * 16
        pltpu.sync_copy(src_hbm.at[pl.ds(base, 16)], sidx)
        pltpu.sync_copy(dst_hbm.at[pl.ds(base, 16)], didx)
        for h in range(2):
            gd = [
                pltpu.async_copy(ts_hbm.at[sidx.at[h * 8 + j]],
                                 rows.at[j], sem1)
                for j in range(8)
            ]
            sd = []
            for j in range(8):
                gd[j].wait()
                sd.append(
                    pltpu.async_copy(rows.at[j], acc.at[didx.at[h * 8 + j]],
                                     sem2, add=True))
            for d in sd:
                d.wait()

    plsc.subcore_barrier()
    pltpu.sync_copy(acc.at[pl.ds(s * STRIPE, STRIPE)],
                    out_hbm.at[c, pl.ds(s * STRIPE, STRIPE)])


@functools.partial(
    pl.kernel,
    out_type=(
        jax.ShapeDtypeStruct((PPAD, FW), _f32),
        jax.ShapeDtypeStruct((PPAD, FW), _f32),
    ),
    mesh=_sc_mesh,
    compiler_params=pltpu.CompilerParams(use_tc_tiling_on_sc=False),
    scratch_types=[
        pltpu.VMEM((PCW, 128), jnp.int32),
        pltpu.VMEM((PCW, 128), jnp.int32),
        pltpu.VMEM((8, 128, FW), _f32),
        pltpu.VMEM((8, 128, FW), _f32),
        pltpu.SemaphoreType.DMA,
        pltpu.SemaphoreType.DMA,
    ],
)
def _pair_kernel(n1_hbm, n2_hbm, u_hbm, v_hbm, a_out, b_out,
                 idx1, idx2, ar, br, sem1, sem2):
    c = lax.axis_index("c")
    s = lax.axis_index("s")
    w = s * NC + c
    pltpu.sync_copy(n1_hbm.at[pl.ds(w * PCW, PCW)], idx1)
    pltpu.sync_copy(n2_hbm.at[pl.ds(w * PCW, PCW)], idx2)

    @pl.loop(0, PCW // 8)
    def _grp(g):
        base = w * PCW + g * 8
        ga = [
            pltpu.async_copy(u_hbm.at[idx1.at[g * 8 + j]], ar.at[j], sem1)
            for j in range(8)
        ]
        gb = [
            pltpu.async_copy(v_hbm.at[idx2.at[g * 8 + j]], br.at[j], sem1)
            for j in range(8)
        ]
        sd = []
        for j in range(8):
            ga[j].wait()
            sd.append(pltpu.async_copy(
                ar.at[j], a_out.at[pl.ds((base + j) * 128, 128)], sem2))
        for j in range(8):
            gb[j].wait()
            sd.append(pltpu.async_copy(
                br.at[j], b_out.at[pl.ds((base + j) * 128, 128)], sem2))
        for d in sd:
            d.wait()


def kernel(x, edge_index, bank, node1, node2, W1, b1, W2, b2, Wb, bb,
           Wf1, bf1, Wf2, bf2, Wf3, bf3):
    src = edge_index[0].astype(jnp.int32)
    dst = edge_index[1].astype(jnp.int32)
    n1 = node1.astype(jnp.int32)
    n2 = node2.astype(jnp.int32)
    n1 = jnp.where(n1 < N_NODES, n1, n1 + (NR - N_NODES))
    n2 = jnp.where(n2 < N_NODES, n2, n2 + (NR - N_NODES))

    Wc = Wf1 @ Wf2 @ Wf3
    bc = bf1 @ Wf2 @ Wf3 + bf2 @ Wf3 + bf3
    Wcc = jnp.concatenate([Wc[:50], Wc[50:]], axis=1)
    W_uv = W1 @ W2 @ Wcc
    cuv = (b1 @ W2) @ Wcc
    duv = b2 @ Wcc
    bank_w = Wb @ Wcc
    bank_b = bb @ Wcc

    srcp = jnp.pad(src, (0, EPAD - N_EDGES),
                   constant_values=DUMMY_NODE).reshape(ECH, 128)
    dstp = jnp.pad(dst, (0, EPAD - N_EDGES),
                   constant_values=DUMMY_NODE).reshape(ECH, 128)
    dummy2 = DUMMY_PAIR + (NR - N_NODES)
    n1p = jnp.pad(n1, (0, PPAD - N_PAIRS),
                  constant_values=dummy2).reshape(PCH, 128)
    n2p = jnp.pad(n2, (0, PPAD - N_PAIRS),
                  constant_values=dummy2).reshape(PCH, 128)

    x32 = jnp.pad(x, ((0, NR - N_NODES), (0, 5)))
    w_pad = jnp.pad(W_uv, ((0, 5), (0, FW - 8)))
    ones_row = jnp.zeros((1, FW), _f32).at[0, 8].set(1.0)
    g = _node_matmul(x32, w_pad, ones_row, STRIPE)

    bank32 = jnp.pad(bank, ((0, NPF2 - NR - N_BANK, ), (0, 5)))
    bw_pad = jnp.pad(bank_w, ((0, 5), (0, FW - 8)))
    bank_bias = jnp.pad(bank_b.at[:4].add(bc), (0, FW - 8))[None, :]
    bank16 = _node_matmul(bank32, bw_pad, bank_bias, STRIPE)
    brow = jnp.arange(NPF2 - NR, dtype=jnp.int32)[:, None]
    bank_tab = jnp.where(brow < N_BANK, bank16, -jnp.inf)

    degp = _deg_kernel(dstp)
    d0 = degp[0].reshape(NR, 1)
    d1 = degp[1].reshape(NR, 1)
    ts1, dis = _row_kernel(_scale1_body, (FW, 1), (d0, d1, g))

    acc1 = _prop_kernel(srcp, dstp, ts1)
    out1, ts2 = _row_kernel(
        _scale2_body, (FW, FW), (acc1[0], acc1[1], g, dis))

    acc2 = _prop_kernel(srcp, dstp, ts2)
    cuv_row = jnp.pad(cuv, (0, FW - 8))[None, :]
    duv_row = jnp.pad(duv.at[:4].add(bc), (0, FW - 8))[None, :]
    q_tab = _scale3_kernel(acc2[0], acc2[1], out1, dis, cuv_row, duv_row,
                           bank_tab)

    a16, b16 = _pair_kernel(n1p, n2p, q_tab, q_tab)
    a2d = a16.reshape(PPAD // 8, 128)
    b2d = b16.reshape(PPAD // 8, 128)
    out2d = _softmax_axis0(a2d, b2d, 1024)
    return out2d[:N_PAIRS // 8].reshape(N_PAIRS, FW)[:, :4]


# speedup vs baseline: 22.4732x; 1.0646x over previous
import functools

import jax
import jax.numpy as jnp
from jax import lax
from jax.experimental import pallas as pl
from jax.experimental.pallas import tpu as pltpu
from jax.experimental.pallas import tpu_sc as plsc

N_NODES = 100000
N_BANK = 10000
N_PAIRS = 500000
N_EDGES = 1600000
FW = 16
NC, NS = 2, 16
NW = NC * NS
NR = 100352
STRIPE = NR // NS
DUMMY_NODE = N_NODES
ECH = 12800
EPAD = ECH * 128
CW = ECH // NW
NPF2 = 112896
DUMMY_PAIR = N_NODES + N_BANK
PCH = 4096
PPAD = PCH * 128
PCW = PCH // NW

_sc_mesh = plsc.VectorSubcoreMesh(core_axis_name="c", subcore_axis_name="s")
_f32 = jnp.float32


def _matmul_body(x_ref, w_ref, c_ref, o_ref):
    o_ref[...] = (
        jnp.dot(x_ref[...], w_ref[...], preferred_element_type=_f32)
        + c_ref[...]
    )


def _node_matmul(x32, w_pad, row_const, bm):
    m = x32.shape[0]
    return pl.pallas_call(
        _matmul_body,
        grid=(m // bm,),
        in_specs=[
            pl.BlockSpec((bm, 32), lambda i: (i, 0)),
            pl.BlockSpec((32, FW), lambda i: (0, 0)),
            pl.BlockSpec((1, FW), lambda i: (0, 0)),
        ],
        out_specs=pl.BlockSpec((bm, FW), lambda i: (i, 0)),
        out_shape=jax.ShapeDtypeStruct((m, FW), _f32),
    )(x32, w_pad, row_const)


def _scale1_body(d0_ref, d1_ref, g_ref, ts_ref, dis_ref):
    dis = lax.rsqrt(d0_ref[...] + d1_ref[...] + 1.0)
    dis_ref[...] = dis
    ts_ref[...] = dis * g_ref[...]


def _scale2_body(a0_ref, a1_ref, g_ref, dis_ref, out1_ref, ts_ref):
    dis = dis_ref[...]
    out1 = dis * (a0_ref[...] + a1_ref[...]) + (dis * dis) * g_ref[...]
    out1_ref[...] = out1
    ts_ref[...] = dis * out1


def _scale3_body(a0_ref, a1_ref, out1_ref, dis_ref, cuv_ref, duv_ref,
                 bank_ref, q_ref):
    i = pl.program_id(0)

    @pl.when(i < 16)
    def _node():
        dis = dis_ref[...]
        out1 = out1_ref[...]
        out2 = dis * (a0_ref[...] + a1_ref[...]) + (dis * dis) * out1
        r = out1[:, 8:9]
        q = out2 + r * cuv_ref[...] + duv_ref[...]
        q_ref[...] = q[:, :8]

    @pl.when(i >= 16)
    def _bank():
        q_ref[...] = bank_ref[...][:, :8]


def _scale3_kernel(a0, a1, out1, dis, cuv_row, duv_row, bank_tab):
    bm = STRIPE
    row16 = pl.BlockSpec((bm, FW), lambda i: (jnp.minimum(i, 15), 0))
    out_spec = pl.BlockSpec((bm, 8), lambda i: (i, 0))
    return pl.pallas_call(
        _scale3_body,
        grid=(18,),
        in_specs=[
            row16, row16,
            row16,
            pl.BlockSpec((bm, 1), lambda i: (jnp.minimum(i, 15), 0)),
            pl.BlockSpec((1, FW), lambda i: (0, 0)),
            pl.BlockSpec((1, FW), lambda i: (0, 0)),
            pl.BlockSpec((bm, FW), lambda i: (jnp.maximum(i - 16, 0), 0)),
        ],
        out_specs=out_spec,
        out_shape=jax.ShapeDtypeStruct((NPF2, 8), _f32),
    )(a0, a1, out1, dis, cuv_row, duv_row, bank_tab)


def _row_kernel(body, n_out, inputs, col_inputs=0, bm=STRIPE):
    m = inputs[0].shape[0]
    specs = []
    for a in inputs:
        if a.shape == (m, FW):
            specs.append(pl.BlockSpec((bm, FW), lambda i: (i, 0)))
        elif a.shape == (m, 1):
            specs.append(pl.BlockSpec((bm, 1), lambda i: (i, 0)))
        else:
            specs.append(pl.BlockSpec((1, FW), lambda i: (0, 0)))
    out_shapes = []
    out_specs = []
    for shp in n_out:
        out_shapes.append(jax.ShapeDtypeStruct((m, shp), _f32))
        out_specs.append(pl.BlockSpec((bm, shp), lambda i: (i, 0)))
    return pl.pallas_call(
        body,
        grid=(m // bm,),
        in_specs=specs,
        out_specs=out_specs,
        out_shape=out_shapes,
    )(*inputs)


def _softmax_body(a_ref, b_ref, o_ref, m_ref, s_ref):
    phase = pl.program_id(0)
    nb = pl.num_programs(1)
    i = pl.program_id(1)

    @pl.when(jnp.logical_and(phase == 0, i == 0))
    def _init():
        m_ref[...] = jnp.full_like(m_ref, -jnp.inf)
        s_ref[...] = jnp.zeros_like(s_ref)

    @pl.when(phase == 0)
    def _acc():
        blk = a_ref[...] + pltpu.roll(b_ref[...], 124, 1)
        bm_ = jnp.max(blk, axis=0, keepdims=True)
        m_old = m_ref[...]
        m_new = jnp.maximum(m_old, bm_)
        bs = jnp.sum(jnp.exp(blk - m_new), axis=0, keepdims=True)
        s_ref[...] = s_ref[...] * jnp.exp(m_old - m_new) + bs
        m_ref[...] = m_new

        @pl.when(i == nb - 1)
        def _lanefold():
            mm = m_ref[...]
            ss = s_ref[...]
            for k in (8, 16, 32, 64):
                mr = pltpu.roll(mm, k, 1)
                sr = pltpu.roll(ss, k, 1)
                mn = jnp.maximum(mm, mr)
                ss = ss * jnp.exp(mm - mn) + sr * jnp.exp(mr - mn)
                mm = mn
            m_ref[...] = mm
            s_ref[...] = ss

    @pl.when(phase == 1)
    def _emit():
        o_ref[...] = (jnp.exp(a_ref[...] + pltpu.roll(b_ref[...], 124, 1)
                              - m_ref[...]) * (1.0 / s_ref[...]))


def _softmax_axis0(a2d, b2d, bm):
    r = a2d.shape[0]
    spec = pl.BlockSpec((bm, 128), lambda p, i: (i, 0))
    return pl.pallas_call(
        _softmax_body,
        grid=(2, r // bm),
        in_specs=[spec, spec],
        out_specs=pl.BlockSpec((bm, 128), lambda p, i: (i, 0)),
        out_shape=jax.ShapeDtypeStruct((r, 128), _f32),
        scratch_shapes=[
            pltpu.VMEM((1, 128), _f32),
            pltpu.VMEM((1, 128), _f32),
        ],
        compiler_params=pltpu.CompilerParams(
            dimension_semantics=("arbitrary", "arbitrary")
        ),
    )(a2d, b2d)


@functools.partial(
    pl.kernel,
    out_type=jax.ShapeDtypeStruct((NC, NR), _f32),
    mesh=_sc_mesh,
    compiler_params=pltpu.CompilerParams(use_tc_tiling_on_sc=False),
    scratch_types=[
        pltpu.VMEM((CW, 128), jnp.int32),
        pltpu.VMEM((128,), _f32),
        pltpu.VMEM((STRIPE,), _f32),
        pltpu.VMEM_SHARED((NR,), _f32),
        pltpu.SemaphoreType.DMA,
    ],
)
def _deg_kernel(dst_hbm, out_hbm, didx, ones_v, zer_v, acc, sem):
    c = lax.axis_index("c")
    s = lax.axis_index("s")
    w = s * NC + c

    @pl.loop(0, STRIPE // 16)
    def _zf(i):
        zer_v[pl.ds(i * 16, 16)] = jnp.zeros((16,), _f32)

    for i in range(8):
        ones_v[pl.ds(i * 16, 16)] = jnp.ones((16,), _f32)

    pltpu.sync_copy(zer_v, acc.at[pl.ds(s * STRIPE, STRIPE)])
    pltpu.sync_copy(dst_hbm.at[pl.ds(w * CW, CW)], didx)
    plsc.subcore_barrier()

    @pl.loop(0, CW // 16)
    def _grp(g):
        descs = [
            pltpu.async_copy(ones_v, acc.at[didx.at[g * 16 + j]], sem,
                             add=True)
            for j in range(16)
        ]
        for d in descs:
            d.wait()

    plsc.subcore_barrier()
    pltpu.sync_copy(acc.at[pl.ds(s * STRIPE, STRIPE)],
                    out_hbm.at[c, pl.ds(s * STRIPE, STRIPE)])


@functools.partial(
    pl.kernel,
    out_type=jax.ShapeDtypeStruct((NC, NR, FW), _f32),
    mesh=_sc_mesh,
    compiler_params=pltpu.CompilerParams(use_tc_tiling_on_sc=False),
    scratch_types=[
        pltpu.VMEM((20, 128), jnp.int32),
        pltpu.VMEM((20, 128), jnp.int32),
        pltpu.VMEM((10, 128, FW), _f32),
        pltpu.VMEM((128, FW), _f32),
        pltpu.VMEM_SHARED((NR, FW), _f32),
        pltpu.SemaphoreType.DMA,
        pltpu.SemaphoreType.DMA,
    ],
)
def _prop_kernel(src_hbm, dst_hbm, ts_hbm, out_hbm,
                 sidx, didx, rows, zer, acc, sem1, sem2):
    c = lax.axis_index("c")
    s = lax.axis_index("s")
    w = s * NC + c

    @pl.loop(0, 128)
    def _zf(i):
        zer[i, :] = jnp.zeros((16,), _f32)

    @pl.loop(0, STRIPE // 128)
    def _zs(k):
        pltpu.sync_copy(zer, acc.at[pl.ds(s * STRIPE + k * 128, 128)])

    plsc.subcore_barrier()

    @pl.loop(0, CW // 20)
    def _sup(g):
        base = w * CW + g * 20
        pltpu.sync_copy(src_hbm.at[pl.ds(base, 20)], sidx)
        pltpu.sync_copy(dst_hbm.at[pl.ds(base, 20)], didx)
        for h in range(2):
            gd = [
                pltpu.async_copy(ts_hbm.at[sidx.at[h * 10 + j]],
                                 rows.at[j], sem1)
                for j in range(10)
            ]
            sd = []
            for j in range(10):
                gd[j].wait()
                sd.append(
                    pltpu.async_copy(rows.at[j], acc.at[didx.at[h * 10 + j]],
                                     sem2, add=True))
            for d in sd:
                d.wait()

    plsc.subcore_barrier()
    pltpu.sync_copy(acc.at[pl.ds(s * STRIPE, STRIPE)],
                    out_hbm.at[c, pl.ds(s * STRIPE, STRIPE)])


@functools.partial(
    pl.kernel,
    out_type=(
        jax.ShapeDtypeStruct((PPAD, 8), _f32),
        jax.ShapeDtypeStruct((PPAD, 8), _f32),
    ),
    mesh=_sc_mesh,
    compiler_params=pltpu.CompilerParams(use_tc_tiling_on_sc=False),
    scratch_types=[
        pltpu.VMEM((PCW, 128), jnp.int32),
        pltpu.VMEM((PCW, 128), jnp.int32),
        pltpu.VMEM((8, 128, 8), _f32),
        pltpu.VMEM((8, 128, 8), _f32),
        pltpu.SemaphoreType.DMA,
        pltpu.SemaphoreType.DMA,
    ],
)
def _pair_kernel(n1_hbm, n2_hbm, u_hbm, v_hbm, a_out, b_out,
                 idx1, idx2, ar, br, sem1, sem2):
    c = lax.axis_index("c")
    s = lax.axis_index("s")
    w = s * NC + c
    pltpu.sync_copy(n1_hbm.at[pl.ds(w * PCW, PCW)], idx1)
    pltpu.sync_copy(n2_hbm.at[pl.ds(w * PCW, PCW)], idx2)

    @pl.loop(0, PCW // 8)
    def _grp(g):
        base = w * PCW + g * 8
        ga = [
            pltpu.async_copy(u_hbm.at[idx1.at[g * 8 + j]], ar.at[j], sem1)
            for j in range(8)
        ]
        gb = [
            pltpu.async_copy(v_hbm.at[idx2.at[g * 8 + j]], br.at[j], sem1)
            for j in range(8)
        ]
        sd = []
        for j in range(8):
            ga[j].wait()
            sd.append(pltpu.async_copy(
                ar.at[j], a_out.at[pl.ds((base + j) * 128, 128)], sem2))
        for j in range(8):
            gb[j].wait()
            sd.append(pltpu.async_copy(
                br.at[j], b_out.at[pl.ds((base + j) * 128, 128)], sem2))
        for d in sd:
            d.wait()


def kernel(x, edge_index, bank, node1, node2, W1, b1, W2, b2, Wb, bb,
           Wf1, bf1, Wf2, bf2, Wf3, bf3):
    src = edge_index[0].astype(jnp.int32)
    dst = edge_index[1].astype(jnp.int32)
    n1 = node1.astype(jnp.int32)
    n2 = node2.astype(jnp.int32)
    n1 = jnp.where(n1 < N_NODES, n1, n1 + (NR - N_NODES))
    n2 = jnp.where(n2 < N_NODES, n2, n2 + (NR - N_NODES))

    Wc = Wf1 @ Wf2 @ Wf3
    bc = bf1 @ Wf2 @ Wf3 + bf2 @ Wf3 + bf3
    Wcc = jnp.concatenate([Wc[:50], Wc[50:]], axis=1)
    W_uv = W1 @ W2 @ Wcc
    cuv = (b1 @ W2) @ Wcc
    duv = b2 @ Wcc
    bank_w = Wb @ Wcc
    bank_b = bb @ Wcc

    srcp = jnp.pad(src, (0, EPAD - N_EDGES),
                   constant_values=DUMMY_NODE).reshape(ECH, 128)
    dstp = jnp.pad(dst, (0, EPAD - N_EDGES),
                   constant_values=DUMMY_NODE).reshape(ECH, 128)
    dummy2 = DUMMY_PAIR + (NR - N_NODES)
    n1p = jnp.pad(n1, (0, PPAD - N_PAIRS),
                  constant_values=dummy2).reshape(PCH, 128)
    n2p = jnp.pad(n2, (0, PPAD - N_PAIRS),
                  constant_values=dummy2).reshape(PCH, 128)

    x32 = jnp.pad(x, ((0, NR - N_NODES), (0, 5)))
    w_pad = jnp.pad(W_uv, ((0, 5), (0, FW - 8)))
    ones_row = jnp.zeros((1, FW), _f32).at[0, 8].set(1.0)
    g = _node_matmul(x32, w_pad, ones_row, STRIPE)

    bank32 = jnp.pad(bank, ((0, NPF2 - NR - N_BANK, ), (0, 5)))
    bw_pad = jnp.pad(bank_w, ((0, 5), (0, FW - 8)))
    bank_bias = jnp.pad(bank_b.at[:4].add(bc), (0, FW - 8))[None, :]
    bank16 = _node_matmul(bank32, bw_pad, bank_bias, STRIPE)
    brow = jnp.arange(NPF2 - NR, dtype=jnp.int32)[:, None]
    bank_tab = jnp.where(brow < N_BANK, bank16, -jnp.inf)

    degp = _deg_kernel(dstp)
    d0 = degp[0].reshape(NR, 1)
    d1 = degp[1].reshape(NR, 1)
    ts1, dis = _row_kernel(_scale1_body, (FW, 1), (d0, d1, g))

    acc1 = _prop_kernel(srcp, dstp, ts1)
    out1, ts2 = _row_kernel(
        _scale2_body, (FW, FW), (acc1[0], acc1[1], g, dis))

    acc2 = _prop_kernel(srcp, dstp, ts2)
    cuv_row = jnp.pad(cuv, (0, FW - 8))[None, :]
    duv_row = jnp.pad(duv.at[:4].add(bc), (0, FW - 8))[None, :]
    q_tab = _scale3_kernel(acc2[0], acc2[1], out1, dis, cuv_row, duv_row,
                           bank_tab)

    a16, b16 = _pair_kernel(n1p, n2p, q_tab, q_tab)
    a2d = a16.reshape(PPAD // 16, 128)
    b2d = b16.reshape(PPAD // 16, 128)
    out2d = _softmax_axis0(a2d, b2d, 1024)
    return out2d[:N_PAIRS // 16].reshape(N_PAIRS, 8)[:, :4]


# speedup vs baseline: 23.0977x; 1.0278x over previous
import functools

import jax
import jax.numpy as jnp
from jax import lax
from jax.experimental import pallas as pl
from jax.experimental.pallas import tpu as pltpu
from jax.experimental.pallas import tpu_sc as plsc

N_NODES = 100000
N_BANK = 10000
N_PAIRS = 500000
N_EDGES = 1600000
FW = 8
NC, NS = 2, 16
NW = NC * NS
NR = 100352
STRIPE = NR // NS
DUMMY_NODE = N_NODES
ECH = 12800
EPAD = ECH * 128
CW = ECH // NW
NPF2 = 112896
DUMMY_PAIR = N_NODES + N_BANK
PCH = 4096
PPAD = PCH * 128
PCW = PCH // NW

_sc_mesh = plsc.VectorSubcoreMesh(core_axis_name="c", subcore_axis_name="s")
_f32 = jnp.float32


def _matmul_body(x_ref, w_ref, c_ref, o_ref):
    o_ref[...] = (
        jnp.dot(x_ref[...], w_ref[...], preferred_element_type=_f32)
        + c_ref[...]
    )


def _node_matmul(x32, w_pad, row_const, bm):
    m = x32.shape[0]
    return pl.pallas_call(
        _matmul_body,
        grid=(m // bm,),
        in_specs=[
            pl.BlockSpec((bm, 32), lambda i: (i, 0)),
            pl.BlockSpec((32, FW), lambda i: (0, 0)),
            pl.BlockSpec((1, FW), lambda i: (0, 0)),
        ],
        out_specs=pl.BlockSpec((bm, FW), lambda i: (i, 0)),
        out_shape=jax.ShapeDtypeStruct((m, FW), _f32),
    )(x32, w_pad, row_const)


def _scale1_body(d0_ref, d1_ref, g_ref, ts_ref, dis_ref):
    dis = lax.rsqrt(d0_ref[...] + d1_ref[...] + 1.0)
    dis_ref[...] = dis
    ts_ref[...] = dis * g_ref[...]


def _scale2_body(a0_ref, a1_ref, g_ref, dis_ref, cuv_ref, y1_ref, ts_ref):
    dis = dis_ref[...]
    y1 = (dis * (a0_ref[...] + a1_ref[...]) + (dis * dis) * g_ref[...]
          + cuv_ref[...])
    y1_ref[...] = y1
    ts_ref[...] = dis * y1


def _scale3_body(a0_ref, a1_ref, y1_ref, dis_ref, duv_ref,
                 bank_ref, q_ref):
    i = pl.program_id(0)

    @pl.when(i < 16)
    def _node():
        dis = dis_ref[...]
        q_ref[...] = (dis * (a0_ref[...] + a1_ref[...])
                      + (dis * dis) * y1_ref[...] + duv_ref[...])

    @pl.when(i >= 16)
    def _bank():
        q_ref[...] = bank_ref[...]


def _scale3_kernel(a0, a1, y1, dis, duv_row, bank_tab):
    bm = STRIPE
    row8 = pl.BlockSpec((bm, FW), lambda i: (jnp.minimum(i, 15), 0))
    return pl.pallas_call(
        _scale3_body,
        grid=(18,),
        in_specs=[
            row8, row8,
            row8,
            pl.BlockSpec((bm, 1), lambda i: (jnp.minimum(i, 15), 0)),
            pl.BlockSpec((1, FW), lambda i: (0, 0)),
            pl.BlockSpec((bm, FW), lambda i: (jnp.maximum(i - 16, 0), 0)),
        ],
        out_specs=pl.BlockSpec((bm, FW), lambda i: (i, 0)),
        out_shape=jax.ShapeDtypeStruct((NPF2, FW), _f32),
    )(a0, a1, y1, dis, duv_row, bank_tab)


def _row_kernel(body, n_out, inputs, col_inputs=0, bm=STRIPE):
    m = inputs[0].shape[0]
    specs = []
    for a in inputs:
        if a.shape == (m, FW):
            specs.append(pl.BlockSpec((bm, FW), lambda i: (i, 0)))
        elif a.shape == (m, 1):
            specs.append(pl.BlockSpec((bm, 1), lambda i: (i, 0)))
        else:
            specs.append(pl.BlockSpec((1, FW), lambda i: (0, 0)))
    out_shapes = []
    out_specs = []
    for shp in n_out:
        out_shapes.append(jax.ShapeDtypeStruct((m, shp), _f32))
        out_specs.append(pl.BlockSpec((bm, shp), lambda i: (i, 0)))
    return pl.pallas_call(
        body,
        grid=(m // bm,),
        in_specs=specs,
        out_specs=out_specs,
        out_shape=out_shapes,
    )(*inputs)


def _softmax_body(a_ref, b_ref, o_ref, m_ref, s_ref):
    phase = pl.program_id(0)
    nb = pl.num_programs(1)
    i = pl.program_id(1)

    @pl.when(jnp.logical_and(phase == 0, i == 0))
    def _init():
        m_ref[...] = jnp.full_like(m_ref, -jnp.inf)
        s_ref[...] = jnp.zeros_like(s_ref)

    @pl.when(phase == 0)
    def _acc():
        blk = a_ref[...] + pltpu.roll(b_ref[...], 124, 1)
        bm_ = jnp.max(blk, axis=0, keepdims=True)
        m_old = m_ref[...]
        m_new = jnp.maximum(m_old, bm_)
        bs = jnp.sum(jnp.exp(blk - m_new), axis=0, keepdims=True)
        s_ref[...] = s_ref[...] * jnp.exp(m_old - m_new) + bs
        m_ref[...] = m_new

        @pl.when(i == nb - 1)
        def _lanefold():
            mm = m_ref[...]
            ss = s_ref[...]
            for k in (8, 16, 32, 64):
                mr = pltpu.roll(mm, k, 1)
                sr = pltpu.roll(ss, k, 1)
                mn = jnp.maximum(mm, mr)
                ss = ss * jnp.exp(mm - mn) + sr * jnp.exp(mr - mn)
                mm = mn
            m_ref[...] = mm
            s_ref[...] = ss

    @pl.when(phase == 1)
    def _emit():
        o_ref[...] = (jnp.exp(a_ref[...] + pltpu.roll(b_ref[...], 124, 1)
                              - m_ref[...]) * (1.0 / s_ref[...]))


def _softmax_axis0(a2d, b2d, bm):
    r = a2d.shape[0]
    spec = pl.BlockSpec((bm, 128), lambda p, i: (i, 0))
    return pl.pallas_call(
        _softmax_body,
        grid=(2, r // bm),
        in_specs=[spec, spec],
        out_specs=pl.BlockSpec((bm, 128), lambda p, i: (i, 0)),
        out_shape=jax.ShapeDtypeStruct((r, 128), _f32),
        scratch_shapes=[
            pltpu.VMEM((1, 128), _f32),
            pltpu.VMEM((1, 128), _f32),
        ],
        compiler_params=pltpu.CompilerParams(
            dimension_semantics=("arbitrary", "arbitrary")
        ),
    )(a2d, b2d)


@functools.partial(
    pl.kernel,
    out_type=jax.ShapeDtypeStruct((NC, NR), _f32),
    mesh=_sc_mesh,
    compiler_params=pltpu.CompilerParams(use_tc_tiling_on_sc=False),
    scratch_types=[
        pltpu.VMEM((CW, 128), jnp.int32),
        pltpu.VMEM((128,), _f32),
        pltpu.VMEM((STRIPE,), _f32),
        pltpu.VMEM_SHARED((NR,), _f32),
        pltpu.SemaphoreType.DMA,
    ],
)
def _deg_kernel(dst_hbm, out_hbm, didx, ones_v, zer_v, acc, sem):
    c = lax.axis_index("c")
    s = lax.axis_index("s")
    w = s * NC + c

    @pl.loop(0, STRIPE // 16)
    def _zf(i):
        zer_v[pl.ds(i * 16, 16)] = jnp.zeros((16,), _f32)

    for i in range(8):
        ones_v[pl.ds(i * 16, 16)] = jnp.ones((16,), _f32)

    pltpu.sync_copy(zer_v, acc.at[pl.ds(s * STRIPE, STRIPE)])
    pltpu.sync_copy(dst_hbm.at[pl.ds(w * CW, CW)], didx)
    plsc.subcore_barrier()

    @pl.loop(0, CW // 16)
    def _grp(g):
        descs = [
            pltpu.async_copy(ones_v, acc.at[didx.at[g * 16 + j]], sem,
                             add=True)
            for j in range(16)
        ]
        for d in descs:
            d.wait()

    plsc.subcore_barrier()
    pltpu.sync_copy(acc.at[pl.ds(s * STRIPE, STRIPE)],
                    out_hbm.at[c, pl.ds(s * STRIPE, STRIPE)])


@functools.partial(
    pl.kernel,
    out_type=jax.ShapeDtypeStruct((NC, NR, FW), _f32),
    mesh=_sc_mesh,
    compiler_params=pltpu.CompilerParams(use_tc_tiling_on_sc=False),
    scratch_types=[
        pltpu.VMEM((20, 128), jnp.int32),
        pltpu.VMEM((20, 128), jnp.int32),
        pltpu.VMEM((10, 128, FW), _f32),
        pltpu.VMEM_SHARED((NR, FW), _f32),
        pltpu.SemaphoreType.DMA,
        pltpu.SemaphoreType.DMA,
    ],
)
def _prop_kernel(src_hbm, dst_hbm, ts_hbm, zeros_hbm, out_hbm,
                 sidx, didx, rows, acc, sem1, sem2):
    c = lax.axis_index("c")
    s = lax.axis_index("s")
    w = s * NC + c
    pltpu.sync_copy(zeros_hbm, acc.at[pl.ds(s * STRIPE, STRIPE)])
    plsc.subcore_barrier()

    @pl.loop(0, CW // 20)
    def _sup(g):
        base = w * CW + g * 20
        pltpu.sync_copy(src_hbm.at[pl.ds(base, 20)], sidx)
        pltpu.sync_copy(dst_hbm.at[pl.ds(base, 20)], didx)
        for h in range(2):
            gd = [
                pltpu.async_copy(ts_hbm.at[sidx.at[h * 10 + j]],
                                 rows.at[j], sem1)
                for j in range(10)
            ]
            sd = []
            for j in range(10):
                gd[j].wait()
                sd.append(
                    pltpu.async_copy(rows.at[j], acc.at[didx.at[h * 10 + j]],
                                     sem2, add=True))
            for d in sd:
                d.wait()

    plsc.subcore_barrier()
    pltpu.sync_copy(acc.at[pl.ds(s * STRIPE, STRIPE)],
                    out_hbm.at[c, pl.ds(s * STRIPE, STRIPE)])


@functools.partial(
    pl.kernel,
    out_type=(
        jax.ShapeDtypeStruct((PPAD, 8), _f32),
        jax.ShapeDtypeStruct((PPAD, 8), _f32),
    ),
    mesh=_sc_mesh,
    compiler_params=pltpu.CompilerParams(use_tc_tiling_on_sc=False),
    scratch_types=[
        pltpu.VMEM((PCW, 128), jnp.int32),
        pltpu.VMEM((PCW, 128), jnp.int32),
        pltpu.VMEM((8, 128, 8), _f32),
        pltpu.VMEM((8, 128, 8), _f32),
        pltpu.SemaphoreType.DMA,
        pltpu.SemaphoreType.DMA,
    ],
)
def _pair_kernel(n1_hbm, n2_hbm, u_hbm, v_hbm, a_out, b_out,
                 idx1, idx2, ar, br, sem1, sem2):
    c = lax.axis_index("c")
    s = lax.axis_index("s")
    w = s * NC + c
    pltpu.sync_copy(n1_hbm.at[pl.ds(w * PCW, PCW)], idx1)
    pltpu.sync_copy(n2_hbm.at[pl.ds(w * PCW, PCW)], idx2)

    @pl.loop(0, PCW // 8)
    def _grp(g):
        base = w * PCW + g * 8
        ga = [
            pltpu.async_copy(u_hbm.at[idx1.at[g * 8 + j]], ar.at[j], sem1)
            for j in range(8)
        ]
        gb = [
            pltpu.async_copy(v_hbm.at[idx2.at[g * 8 + j]], br.at[j], sem1)
            for j in range(8)
        ]
        sd = []
        for j in range(8):
            ga[j].wait()
            sd.append(pltpu.async_copy(
                ar.at[j], a_out.at[pl.ds((base + j) * 128, 128)], sem2))
        for j in range(8):
            gb[j].wait()
            sd.append(pltpu.async_copy(
                br.at[j], b_out.at[pl.ds((base + j) * 128, 128)], sem2))
        for d in sd:
            d.wait()


def kernel(x, edge_index, bank, node1, node2, W1, b1, W2, b2, Wb, bb,
           Wf1, bf1, Wf2, bf2, Wf3, bf3):
    src = edge_index[0].astype(jnp.int32)
    dst = edge_index[1].astype(jnp.int32)
    n1 = node1.astype(jnp.int32)
    n2 = node2.astype(jnp.int32)
    n1 = jnp.where(n1 < N_NODES, n1, n1 + (NR - N_NODES))
    n2 = jnp.where(n2 < N_NODES, n2, n2 + (NR - N_NODES))

    Wc = Wf1 @ Wf2 @ Wf3
    bc = bf1 @ Wf2 @ Wf3 + bf2 @ Wf3 + bf3
    Wcc = jnp.concatenate([Wc[:50], Wc[50:]], axis=1)
    W_uv = W1 @ W2 @ Wcc
    cuv = (b1 @ W2) @ Wcc
    duv = b2 @ Wcc
    bank_w = Wb @ Wcc
    bank_b = bb @ Wcc

    srcp = jnp.pad(src, (0, EPAD - N_EDGES),
                   constant_values=DUMMY_NODE).reshape(ECH, 128)
    dstp = jnp.pad(dst, (0, EPAD - N_EDGES),
                   constant_values=DUMMY_NODE).reshape(ECH, 128)
    dummy2 = DUMMY_PAIR + (NR - N_NODES)
    n1p = jnp.pad(n1, (0, PPAD - N_PAIRS),
                  constant_values=dummy2).reshape(PCH, 128)
    n2p = jnp.pad(n2, (0, PPAD - N_PAIRS),
                  constant_values=dummy2).reshape(PCH, 128)

    x32 = jnp.pad(x, ((0, NR - N_NODES), (0, 5)))
    w_pad = jnp.pad(W_uv, ((0, 5), (0, 0)))
    zero_row = jnp.zeros((1, FW), _f32)
    g = _node_matmul(x32, w_pad, zero_row, STRIPE)

    bank32 = jnp.pad(bank, ((0, NPF2 - NR - N_BANK, ), (0, 5)))
    bw_pad = jnp.pad(bank_w, ((0, 5), (0, 0)))
    bank_bias = bank_b.at[:4].add(bc)[None, :]
    bank8 = _node_matmul(bank32, bw_pad, bank_bias, STRIPE)
    brow = jnp.arange(NPF2 - NR, dtype=jnp.int32)[:, None]
    bank_tab = jnp.where(brow < N_BANK, bank8, -jnp.inf)

    degp = _deg_kernel(dstp)
    d0 = degp[0].reshape(NR, 1)
    d1 = degp[1].reshape(NR, 1)
    ts1, dis = _row_kernel(_scale1_body, (FW, 1), (d0, d1, g))

    cuv_row = cuv[None, :]
    zrows = jnp.zeros((STRIPE, FW), _f32)
    acc1 = _prop_kernel(srcp, dstp, ts1, zrows)
    y1, ts2 = _row_kernel(
        _scale2_body, (FW, FW), (acc1[0], acc1[1], g, dis, cuv_row))

    acc2 = _prop_kernel(srcp, dstp, ts2, zrows)
    duv_row = duv.at[:4].add(bc)[None, :]
    q_tab = _scale3_kernel(acc2[0], acc2[1], y1, dis, duv_row, bank_tab)

    a16, b16 = _pair_kernel(n1p, n2p, q_tab, q_tab)
    a2d = a16.reshape(PPAD // 16, 128)
    b2d = b16.reshape(PPAD // 16, 128)
    out2d = _softmax_axis0(a2d, b2d, 1024)
    return out2d[:N_PAIRS // 16].reshape(N_PAIRS, 8)[:, :4]
